# baseline (speedup 1.0000x reference)
import jax
import jax.numpy as jnp
from jax import lax
from jax.experimental import pallas as pl
from jax.experimental.pallas import tpu as pltpu
from jax.experimental.pallas import tpu_sc as plsc

VOCAB = 100000
SEQ_LEN = 50
EMBED = 64
BATCH = 4096

_NC = 2
_NS = 16
_NW = _NC * _NS

_B_TOTAL = BATCH * SEQ_LEN
_B_PER_W = _B_TOTAL // _NW
_CHUNK = 128
_N_CHUNKS = _B_PER_W // _CHUNK
_K = 5
_BIG = _K * _CHUNK
_N_BIG = _N_CHUNKS // _K


def _emb_kernel(x_hbm, table_hbm, out_hbm, idx_v, rows_v,
                gsem0, gsem1, ssem0, ssem1):
    wid = lax.axis_index("s") * _NC + lax.axis_index("c")
    base = wid * _B_PER_W
    pltpu.sync_copy(x_hbm.at[wid], idx_v)

    gsems = (gsem0, gsem1)
    ssems = (ssem0, ssem1)

    def fire(j, b):
        for t in range(_K):
            pltpu.async_copy(
                table_hbm.at[idx_v.at[j * _K + t]],
                rows_v.at[b, pl.ds(t * _CHUNK, _CHUNK)],
                gsems[b],
            )

    def drain(j, b):
        for t in range(_K):
            pltpu.make_async_copy(
                table_hbm.at[idx_v.at[j * _K + t]],
                rows_v.at[b, pl.ds(t * _CHUNK, _CHUNK)],
                gsems[b],
            ).wait()

    def store_fire(j, b):
        pltpu.async_copy(
            rows_v.at[b], out_hbm.at[pl.ds(base + j * _BIG, _BIG)], ssems[b]
        )

    def store_wait(j, b):
        pltpu.make_async_copy(
            rows_v.at[b], out_hbm.at[pl.ds(base + j * _BIG, _BIG)], ssems[b]
        ).wait()

    fire(0, 0)
    for i in range(_N_BIG):
        b = i & 1
        if i + 1 < _N_BIG:
            if i >= 1:
                store_wait(i - 1, 1 - b)
            fire(i + 1, 1 - b)
        drain(i, b)
        store_fire(i, b)
    store_wait(_N_BIG - 2, (_N_BIG - 2) & 1)
    store_wait(_N_BIG - 1, (_N_BIG - 1) & 1)


_TR_P = 256


def _tr_kernel(a_ref, z_ref):
    a = a_ref[0].reshape(_TR_P, 2, EMBED)
    z_ref[0] = a.transpose(2, 0, 1).reshape(EMBED, 2 * _TR_P)


def _to_feature_major(out_sb):
    a = out_sb.reshape(SEQ_LEN, BATCH // 2, 2 * EMBED)
    n_c = BATCH // (2 * _TR_P)
    return pl.pallas_call(
        _tr_kernel,
        grid=(SEQ_LEN, n_c),
        in_specs=[pl.BlockSpec((1, _TR_P, 2 * EMBED), lambda s, c: (s, c, 0))],
        out_specs=pl.BlockSpec((1, EMBED, 2 * _TR_P), lambda s, c: (s, 0, c)),
        out_shape=jax.ShapeDtypeStruct((SEQ_LEN, EMBED, BATCH), jnp.float32),
    )(a)


@jax.jit
def kernel(x, table):
    x_flat = x.T.reshape(_NW, _N_CHUNKS, _CHUNK).astype(jnp.int32)
    mesh = plsc.VectorSubcoreMesh(core_axis_name="c", subcore_axis_name="s")
    out = pl.kernel(
        _emb_kernel,
        mesh=mesh,
        out_type=jax.ShapeDtypeStruct((_B_TOTAL, EMBED), jnp.float32),
        scratch_types=[
            pltpu.VMEM((_N_CHUNKS, _CHUNK), jnp.int32),
            pltpu.VMEM((2, _BIG, EMBED), jnp.float32),
            pltpu.SemaphoreType.DMA,
            pltpu.SemaphoreType.DMA,
            pltpu.SemaphoreType.DMA,
            pltpu.SemaphoreType.DMA,
        ],
        compiler_params=pltpu.CompilerParams(use_tc_tiling_on_sc=False),
    )(x_flat, table)
    zp = _to_feature_major(out)
    return zp.transpose(2, 0, 1)


# speedup vs baseline: 7.3487x; 7.3487x over previous
import jax
import jax.numpy as jnp
from jax import lax
from jax.experimental import pallas as pl
from jax.experimental.pallas import tpu as pltpu
from jax.experimental.pallas import tpu_sc as plsc

VOCAB = 100000
SEQ_LEN = 50
EMBED = 64
BATCH = 4096

_NC = 2
_NS = 16
_NW = _NC * _NS

_B_TOTAL = BATCH * SEQ_LEN
_B_PER_W = _B_TOTAL // _NW
_CHUNK = 128
_N_CHUNKS = _B_PER_W // _CHUNK
_K = 5
_BIG = _K * _CHUNK
_N_BIG = _N_CHUNKS // _K

_HALF = BATCH // 2


def _emb_kernel(x_hbm, table_hbm, out_hbm, idx_v, rows_v,
                gsem0, gsem1, ssem0, ssem1):
    wid = lax.axis_index("s") * _NC + lax.axis_index("c")
    base = wid * _B_PER_W
    pltpu.sync_copy(x_hbm.at[wid], idx_v)

    gsems = (gsem0, gsem1)
    ssems = (ssem0, ssem1)

    def fire(j, b):
        for t in range(_K):
            pltpu.async_copy(
                table_hbm.at[idx_v.at[j * _K + t]],
                rows_v.at[b, pl.ds(t * _CHUNK, _CHUNK)],
                gsems[b],
            )

    def drain(j, b):
        for t in range(_K):
            pltpu.make_async_copy(
                table_hbm.at[idx_v.at[j * _K + t]],
                rows_v.at[b, pl.ds(t * _CHUNK, _CHUNK)],
                gsems[b],
            ).wait()

    def chunk_dst(j, t):
        t0 = base + j * _BIG + t * _CHUNK
        row0 = (t0 // BATCH) * _HALF + t0 % _HALF
        col0 = ((t0 % BATCH) // _HALF) * EMBED
        return out_hbm.at[pl.ds(row0, _CHUNK), pl.ds(col0, EMBED)]

    def store_fire(j, b):
        for t in range(_K):
            pltpu.async_copy(
                rows_v.at[b, pl.ds(t * _CHUNK, _CHUNK)],
                chunk_dst(j, t),
                ssems[b],
            )

    def store_wait(j, b):
        for t in range(_K):
            pltpu.make_async_copy(
                rows_v.at[b, pl.ds(t * _CHUNK, _CHUNK)],
                chunk_dst(j, t),
                ssems[b],
            ).wait()

    fire(0, 0)
    for i in range(_N_BIG):
        b = i & 1
        if i + 1 < _N_BIG:
            if i >= 1:
                store_wait(i - 1, 1 - b)
            fire(i + 1, 1 - b)
        drain(i, b)
        store_fire(i, b)
    store_wait(_N_BIG - 2, (_N_BIG - 2) & 1)
    store_wait(_N_BIG - 1, (_N_BIG - 1) & 1)


_TR_P = 1024


def _tr_kernel(a_ref, z_ref):
    h = pl.program_id(1)
    row = lax.broadcasted_iota(jnp.int32, (EMBED, 2 * EMBED), 0)
    col = lax.broadcasted_iota(jnp.int32, (EMBED, 2 * EMBED), 1)
    sel = (col == h * EMBED + row).astype(jnp.float32)
    z_ref[0] = lax.dot_general(
        sel, a_ref[0],
        dimension_numbers=(((1,), (1,)), ((), ())),
        preferred_element_type=jnp.float32,
    )


def _to_feature_major(out2):
    a = out2.reshape(SEQ_LEN, _HALF, 2 * EMBED)
    n_c = _HALF // _TR_P
    return pl.pallas_call(
        _tr_kernel,
        grid=(SEQ_LEN, 2, n_c),
        in_specs=[
            pl.BlockSpec((1, _TR_P, 2 * EMBED), lambda s, h, c: (s, c, 0))
        ],
        out_specs=pl.BlockSpec(
            (1, EMBED, _TR_P), lambda s, h, c: (s, 0, h * n_c + c)
        ),
        out_shape=jax.ShapeDtypeStruct((SEQ_LEN, EMBED, BATCH), jnp.float32),
    )(a)


@jax.jit
def kernel(x, table):
    x_flat = x.T.reshape(_NW, _N_CHUNKS, _CHUNK).astype(jnp.int32)
    mesh = plsc.VectorSubcoreMesh(core_axis_name="c", subcore_axis_name="s")
    out2 = pl.kernel(
        _emb_kernel,
        mesh=mesh,
        out_type=jax.ShapeDtypeStruct((SEQ_LEN * _HALF, 2 * EMBED),
                                      jnp.float32),
        scratch_types=[
            pltpu.VMEM((_N_CHUNKS, _CHUNK), jnp.int32),
            pltpu.VMEM((2, _BIG, EMBED), jnp.float32),
            pltpu.SemaphoreType.DMA,
            pltpu.SemaphoreType.DMA,
            pltpu.SemaphoreType.DMA,
            pltpu.SemaphoreType.DMA,
        ],
        compiler_params=pltpu.CompilerParams(use_tc_tiling_on_sc=False),
    )(x_flat, table)
    zp = _to_feature_major(out2)
    return zp.transpose(2, 0, 1)


# speedup vs baseline: 7.9314x; 1.0793x over previous
import jax
import jax.numpy as jnp
from jax import lax
from jax.experimental import pallas as pl
from jax.experimental.pallas import tpu as pltpu
from jax.experimental.pallas import tpu_sc as plsc

VOCAB = 100000
SEQ_LEN = 50
EMBED = 64
BATCH = 4096

_NC = 2
_NS = 16
_NW = _NC * _NS

_B_TOTAL = BATCH * SEQ_LEN
_B_PER_W = _B_TOTAL // _NW
_CHUNK = 128
_N_CHUNKS = _B_PER_W // _CHUNK
_K = 5
_BIG = _K * _CHUNK
_N_BIG = _N_CHUNKS // _K


def _emb_kernel(x_hbm, table_hbm, out_hbm, idx_v, rows_v, sem0, sem1):
    wid = lax.axis_index("s") * _NC + lax.axis_index("c")
    base = wid * _B_PER_W
    pltpu.sync_copy(x_hbm.at[pl.ds(base, _B_PER_W)], idx_v)

    sems = (sem0, sem1)

    def fire(j, b):
        for t in range(_K):
            pltpu.async_copy(
                table_hbm.at[idx_v.at[pl.ds((j * _K + t) * _CHUNK, _CHUNK)]],
                rows_v.at[b, pl.ds(t * _CHUNK, _CHUNK)],
                sems[b],
            )

    def drain(j, b):
        for t in range(_K):
            pltpu.make_async_copy(
                table_hbm.at[idx_v.at[pl.ds((j * _K + t) * _CHUNK, _CHUNK)]],
                rows_v.at[b, pl.ds(t * _CHUNK, _CHUNK)],
                sems[b],
            ).wait()

    fire(0, 0)

    def outer(g, carry):
        for b in range(2):
            i = 2 * g + b
            drain(i, b)

            @pl.when(i + 1 < _N_BIG)
            def _():
                fire(i + 1, 1 - b)

            pltpu.sync_copy(
                rows_v.at[b], out_hbm.at[pl.ds(base + i * _BIG, _BIG)]
            )
        return carry

    lax.fori_loop(0, _N_BIG // 2, outer, 0)


@jax.jit
def kernel(x, table):
    x_flat = x.reshape(_B_TOTAL).astype(jnp.int32)
    mesh = plsc.VectorSubcoreMesh(core_axis_name="c", subcore_axis_name="s")
    out = pl.kernel(
        _emb_kernel,
        mesh=mesh,
        out_type=jax.ShapeDtypeStruct((_B_TOTAL, EMBED), jnp.float32),
        scratch_types=[
            pltpu.VMEM((_B_PER_W,), jnp.int32),
            pltpu.VMEM((2, _BIG, EMBED), jnp.float32),
            pltpu.SemaphoreType.DMA,
            pltpu.SemaphoreType.DMA,
        ],
        compiler_params=pltpu.CompilerParams(use_tc_tiling_on_sc=False),
    )(x_flat, table)
    return out.reshape(BATCH, SEQ_LEN, EMBED)
